# baseline (speedup 1.0000x reference)
import jax
import jax.numpy as jnp
from jax import lax
from jax.experimental import pallas as pl
from jax.experimental.pallas import tpu as pltpu


_G = 64


def _body(lens_sref, lens3_ref, x_any, out_ref, scratch, sem):
    k = pl.program_id(0)
    n = pl.num_programs(0)
    g, b_dim, two_d = out_ref.shape
    B = b_dim

    def fetch(kk, slot, op):
        for b in range(B):
            cond = jnp.logical_and(kk < n, kk * g < lens_sref[b])

            @pl.when(cond)
            def _():
                cp = pltpu.make_async_copy(
                    x_any.at[pl.ds(kk * g, g), :, b, :],
                    scratch.at[slot, :, :, b, :],
                    sem,
                )
                if op == "start":
                    cp.start()
                else:
                    cp.wait()

    @pl.when(k == 0)
    def _():
        fetch(0, 0, "start")

    fetch(k + 1, (k + 1) % 2, "start")
    fetch(k, k % 2, "wait")

    def compute(slot):
        merged = jnp.concatenate(
            [scratch[slot, :, 0], scratch[slot, :, 1]], axis=-1
        )
        t2 = k * g + lax.broadcasted_iota(jnp.int32, (g, B, two_d), 0)
        lens3 = jnp.broadcast_to(lens3_ref[...], (g, B, two_d))
        out_ref[...] = jnp.where(t2 < lens3, merged, 0.0)

    @pl.when(k % 2 == 0)
    def _():
        compute(0)

    @pl.when(k % 2 == 1)
    def _():
        compute(1)


def kernel(data, lengths):
    T, B, D = data.shape
    T2 = T - (T % 2)
    H = T2 // 2
    newlens = (lengths // 2).astype(jnp.int32)
    x = data[:T2].reshape(H, 2, B, D)
    lens3d = newlens.reshape(1, B, 1)

    grid_spec = pltpu.PrefetchScalarGridSpec(
        num_scalar_prefetch=1,
        grid=(H // _G,),
        in_specs=[
            pl.BlockSpec((1, B, 1), lambda k, lens: (0, 0, 0)),
            pl.BlockSpec(memory_space=pl.ANY),
        ],
        out_specs=pl.BlockSpec((_G, B, 2 * D), lambda k, lens: (k, 0, 0)),
        scratch_shapes=[
            pltpu.VMEM((2, _G, 2, B, D), jnp.float32),
            pltpu.SemaphoreType.DMA,
        ],
    )
    out = pl.pallas_call(
        _body,
        grid_spec=grid_spec,
        out_shape=jax.ShapeDtypeStruct((H, B, 2 * D), data.dtype),
    )(newlens, lens3d, x)
    return out, newlens


# speedup vs baseline: 1.0487x; 1.0487x over previous
import jax
import jax.numpy as jnp
from jax import lax
from jax.experimental import pallas as pl
from jax.experimental.pallas import tpu as pltpu


_G = 128


def _body(lens_sref, lens3_ref, x_any, out_ref, scratch, sem):
    k = pl.program_id(0)
    n = pl.num_programs(0)
    g, b_dim, two_d = out_ref.shape
    B = b_dim

    def fetch(kk, slot, op):
        for b in range(B):
            cond = jnp.logical_and(kk < n, kk * g < lens_sref[b])

            @pl.when(cond)
            def _():
                cp = pltpu.make_async_copy(
                    x_any.at[pl.ds(kk * g, g), :, b, :],
                    scratch.at[slot, :, :, b, :],
                    sem,
                )
                if op == "start":
                    cp.start()
                else:
                    cp.wait()

    @pl.when(k == 0)
    def _():
        fetch(0, 0, "start")

    fetch(k + 1, (k + 1) % 2, "start")
    fetch(k, k % 2, "wait")

    def compute(slot):
        merged = jnp.concatenate(
            [scratch[slot, :, 0], scratch[slot, :, 1]], axis=-1
        )
        t2 = k * g + lax.broadcasted_iota(jnp.int32, (g, B, two_d), 0)
        lens3 = jnp.broadcast_to(lens3_ref[...], (g, B, two_d))
        out_ref[...] = jnp.where(t2 < lens3, merged, 0.0)

    @pl.when(k % 2 == 0)
    def _():
        compute(0)

    @pl.when(k % 2 == 1)
    def _():
        compute(1)


def kernel(data, lengths):
    T, B, D = data.shape
    T2 = T - (T % 2)
    H = T2 // 2
    newlens = (lengths // 2).astype(jnp.int32)
    x = data[:T2].reshape(H, 2, B, D)
    lens3d = newlens.reshape(1, B, 1)

    grid_spec = pltpu.PrefetchScalarGridSpec(
        num_scalar_prefetch=1,
        grid=(H // _G,),
        in_specs=[
            pl.BlockSpec((1, B, 1), lambda k, lens: (0, 0, 0)),
            pl.BlockSpec(memory_space=pl.ANY),
        ],
        out_specs=pl.BlockSpec((_G, B, 2 * D), lambda k, lens: (k, 0, 0)),
        scratch_shapes=[
            pltpu.VMEM((2, _G, 2, B, D), jnp.float32),
            pltpu.SemaphoreType.DMA,
        ],
    )
    out = pl.pallas_call(
        _body,
        grid_spec=grid_spec,
        out_shape=jax.ShapeDtypeStruct((H, B, 2 * D), data.dtype),
    )(newlens, lens3d, x)
    return out, newlens
